# baseline (speedup 1.0000x reference)
import functools

import jax
import jax.numpy as jnp
from jax import lax
from jax.experimental import pallas as pl
from jax.experimental.pallas import tpu as pltpu
from jax.experimental.pallas import tpu_sc as plsc

VOCAB = 1000000
EMBED = 64
MAX_LEN = 200
BATCH = 4096
PAD_IDX = 0
EPS = 1e-8

NUM_CORES = 2
NUM_SUBCORES = 16
LANES = 16
NW = NUM_CORES * NUM_SUBCORES
B_PER_W = BATCH // NW
BSUB = B_PER_W // LANES
LCHUNK = 5
NCHUNKS = MAX_LEN // LCHUNK
ROWS = B_PER_W * LCHUNK
INV_EMBED = 1.0 / EMBED
NVEC = EMBED // LANES
DT = EMBED // 8


def _rsqrt(z):
    i = plsc.bitcast(z, jnp.int32)
    y = plsc.bitcast(jnp.int32(0x5F3759DF) - (i >> 1), jnp.float32)
    for _ in range(2):
        y = y * (1.5 - 0.5 * z * y * y)
    return y


def _make_kernel():
    mesh = plsc.VectorSubcoreMesh(core_axis_name="c", subcore_axis_name="s")

    @functools.partial(
        pl.kernel,
        mesh=mesh,
        compiler_params=pltpu.CompilerParams(
            needs_layout_passes=False, use_tc_tiling_on_sc=False
        ),
        out_type=jax.ShapeDtypeStruct(
            (MAX_LEN * DT, NW * 8, 128), jnp.float32
        ),
        scratch_types=[
            pltpu.VMEM((ROWS,), jnp.int32),
            pltpu.VMEM((ROWS,), jnp.int32),
            pltpu.VMEM((ROWS, EMBED), jnp.float32),
            pltpu.VMEM((ROWS, EMBED), jnp.float32),
            pltpu.VMEM((LCHUNK * 8, 8, 128), jnp.float32),
            pltpu.VMEM((LCHUNK, EMBED), jnp.float32),
            pltpu.VMEM((LCHUNK, EMBED), jnp.float32),
            pltpu.VMEM((EMBED,), jnp.float32),
            pltpu.VMEM((EMBED,), jnp.float32),
            pltpu.SemaphoreType.DMA,
            pltpu.SemaphoreType.DMA,
            pltpu.SemaphoreType.DMA,
            pltpu.SemaphoreType.DMA,
            pltpu.SemaphoreType.DMA,
        ],
    )
    def emb_kernel(tok_r, w_word, w_pos, gamma, beta, out,
                   idx0, idx1, rows0, rows1, ostage, pos0, pos1,
                   gamma_v, beta_v, gsem0, gsem1, psem0, psem1, osem):
        wid = lax.axis_index("s") * NUM_CORES + lax.axis_index("c")
        lane = lax.iota(jnp.int32, LANES)
        iota_div8 = lane >> 3
        iota_mod8 = lane & 7
        idx_b = (idx0, idx1)
        rows_b = (rows0, rows1)
        pos_b = (pos0, pos1)
        gsem_b = (gsem0, gsem1)
        psem_b = (psem0, psem1)

        pltpu.sync_copy(gamma, gamma_v)
        pltpu.sync_copy(beta, beta_v)
        g4 = [gamma_v[pl.ds(LANES * k, LANES)] for k in range(NVEC)]
        b4 = [beta_v[pl.ds(LANES * k, LANES)] for k in range(NVEC)]

        def tok_off(ci):
            return pl.multiple_of((wid * NCHUNKS + ci) * ROWS, 8)

        def pos_copy(ci, p):
            return pltpu.make_async_copy(
                w_pos.at[pl.ds(ci * LCHUNK, LCHUNK), :], pos_b[p], psem_b[p]
            )

        def stage_and_gather(ci, p):
            pltpu.sync_copy(tok_r.at[pl.ds(tok_off(ci), ROWS)], idx_b[p])
            pltpu.make_async_copy(
                w_word.at[idx_b[p]], rows_b[p], gsem_b[p]
            ).start()
            pos_copy(ci, p).start()

        def wait_gather(ci, p):
            pltpu.make_async_copy(
                w_word.at[idx_b[p]], rows_b[p], gsem_b[p]
            ).wait()
            pos_copy(ci, p).wait()

        def out_copy(ci):
            return pltpu.make_async_copy(
                ostage,
                out.at[
                    pl.ds(ci * LCHUNK * 8, LCHUNK * 8),
                    pl.ds(wid * 8, 8),
                    pl.ds(0, 128),
                ],
                osem,
            )

        def compute(ci, p):
            rows_v = rows_b[p]
            idx_v = idx_b[p]
            pos_v = pos_b[p]

            def t_body(t, carry):
                l = t // BSUB
                bs = t % BSUB
                pos4 = [pos_v[l, pl.ds(LANES * k, LANES)]
                        for k in range(NVEC)]
                tokv = plsc.load_gather(
                    idx_v, [bs * (LANES * LCHUNK) + lane * LCHUNK + l]
                )
                maskv = jnp.where(tokv != PAD_IDX, 1.0, 0.0)
                rowvs = [
                    jnp.full((LANES,), l * 8 + 2 * k, dtype=jnp.int32)
                    + iota_div8
                    for k in range(NVEC)
                ]

                for i in range(LANES):
                    r = (bs * LANES + i) * LCHUNK + l
                    x = [rows_v[r, pl.ds(LANES * k, LANES)] + pos4[k]
                         for k in range(NVEC)]
                    s = (x[0] + x[1]) + (x[2] + x[3])
                    q = (x[0] * x[0] + x[1] * x[1]) + (x[2] * x[2] + x[3] * x[3])
                    mean = jnp.sum(s) * INV_EMBED
                    var = jnp.sum(q) * INV_EMBED - mean * mean
                    z = jnp.full((LANES,), var + EPS, dtype=jnp.float32)
                    rstd = _rsqrt(z)
                    mf = maskv[i]
                    a = rstd * mf
                    iv = jnp.full((LANES,), bs * LANES + i, dtype=jnp.int32)
                    for k in range(NVEC):
                        y = (x[k] - mean) * a * g4[k] + b4[k] * mf
                        plsc.store_scatter(
                            ostage, [rowvs[k], iota_mod8, iv], y
                        )
                return carry
            lax.fori_loop(0, LCHUNK * BSUB, t_body, 0)

        stage_and_gather(0, 0)

        def pair_body(h, carry):
            for p in (0, 1):
                ci = h * 2 + p
                wait_gather(ci, p)

                @pl.when(ci + 1 < NCHUNKS)
                def _prep():
                    stage_and_gather(ci + 1, 1 - p)

                @pl.when(ci > 0)
                def _drain():
                    out_copy(ci - 1).wait()

                compute(ci, p)
                out_copy(ci).start()
            return carry

        lax.fori_loop(0, NCHUNKS // 2, pair_body, 0)
        out_copy(NCHUNKS - 1).wait()

    return emb_kernel


_EMB_KERNEL = _make_kernel()


def kernel(tokens, W_word, W_pos, ln_gamma, ln_beta):
    tok = tokens.astype(jnp.int32)
    tok_r = (
        tok.reshape(NW, B_PER_W, NCHUNKS, LCHUNK)
        .transpose(0, 2, 1, 3)
        .reshape(-1)
    )
    out3 = _EMB_KERNEL(tok_r, W_word, W_pos, ln_gamma, ln_beta)
    return (
        out3.reshape(MAX_LEN, DT, NW, 8, 128)
        .transpose(2, 4, 0, 1, 3)
        .reshape(BATCH, MAX_LEN, EMBED)
    )


# speedup vs baseline: 1.3218x; 1.3218x over previous
import functools

import jax
import jax.numpy as jnp
from jax import lax
from jax.experimental import pallas as pl
from jax.experimental.pallas import tpu as pltpu
from jax.experimental.pallas import tpu_sc as plsc

VOCAB = 1000000
EMBED = 64
MAX_LEN = 200
BATCH = 4096
PAD_IDX = 0
EPS = 1e-8

NUM_CORES = 2
NUM_SUBCORES = 16
LANES = 16
NW = NUM_CORES * NUM_SUBCORES
B_PER_W = BATCH // NW
BSUB = B_PER_W // LANES
LCHUNK = 5
NCHUNKS = MAX_LEN // LCHUNK
ROWS = B_PER_W * LCHUNK
INV_EMBED = 1.0 / EMBED
NVEC = EMBED // LANES
DT = EMBED // 8


def _rsqrt(z):
    i = plsc.bitcast(z, jnp.int32)
    y = plsc.bitcast(jnp.int32(0x5F3759DF) - (i >> 1), jnp.float32)
    for _ in range(2):
        y = y * (1.5 - 0.5 * z * y * y)
    return y


def _make_kernel():
    mesh = plsc.VectorSubcoreMesh(core_axis_name="c", subcore_axis_name="s")

    @functools.partial(
        pl.kernel,
        mesh=mesh,
        compiler_params=pltpu.CompilerParams(
            needs_layout_passes=False, use_tc_tiling_on_sc=False
        ),
        out_type=jax.ShapeDtypeStruct(
            (MAX_LEN * DT, NW * 8, 128), jnp.float32
        ),
        scratch_types=[
            pltpu.VMEM((ROWS,), jnp.int32),
            pltpu.VMEM((ROWS,), jnp.int32),
            pltpu.VMEM((ROWS, EMBED), jnp.float32),
            pltpu.VMEM((ROWS, EMBED), jnp.float32),
            pltpu.VMEM((LCHUNK * 8, 8, 129), jnp.float32),
            pltpu.VMEM((LCHUNK, EMBED), jnp.float32),
            pltpu.VMEM((LCHUNK, EMBED), jnp.float32),
            pltpu.VMEM((EMBED,), jnp.float32),
            pltpu.VMEM((EMBED,), jnp.float32),
            pltpu.SemaphoreType.DMA,
            pltpu.SemaphoreType.DMA,
            pltpu.SemaphoreType.DMA,
            pltpu.SemaphoreType.DMA,
            pltpu.SemaphoreType.DMA,
        ],
    )
    def emb_kernel(tok_r, w_word, w_pos, gamma, beta, out,
                   idx0, idx1, rows0, rows1, ostage, pos0, pos1,
                   gamma_v, beta_v, gsem0, gsem1, psem0, psem1, osem):
        wid = lax.axis_index("s") * NUM_CORES + lax.axis_index("c")
        lane = lax.iota(jnp.int32, LANES)
        iota_div8 = lane >> 3
        iota_mod8 = lane & 7
        idx_b = (idx0, idx1)
        rows_b = (rows0, rows1)
        pos_b = (pos0, pos1)
        gsem_b = (gsem0, gsem1)
        psem_b = (psem0, psem1)

        pltpu.sync_copy(gamma, gamma_v)
        pltpu.sync_copy(beta, beta_v)
        g4 = [gamma_v[pl.ds(LANES * k, LANES)] for k in range(NVEC)]
        b4 = [beta_v[pl.ds(LANES * k, LANES)] for k in range(NVEC)]

        def tok_off(ci):
            return pl.multiple_of((wid * NCHUNKS + ci) * ROWS, 8)

        def pos_copy(ci, p):
            return pltpu.make_async_copy(
                w_pos.at[pl.ds(ci * LCHUNK, LCHUNK), :], pos_b[p], psem_b[p]
            )

        def stage_and_gather(ci, p):
            pltpu.sync_copy(tok_r.at[pl.ds(tok_off(ci), ROWS)], idx_b[p])
            pltpu.make_async_copy(
                w_word.at[idx_b[p]], rows_b[p], gsem_b[p]
            ).start()
            pos_copy(ci, p).start()

        def wait_gather(ci, p):
            pltpu.make_async_copy(
                w_word.at[idx_b[p]], rows_b[p], gsem_b[p]
            ).wait()
            pos_copy(ci, p).wait()

        def out_copy(ci):
            return pltpu.make_async_copy(
                ostage.at[:, :, pl.ds(0, 128)],
                out.at[
                    pl.ds(ci * LCHUNK * 8, LCHUNK * 8),
                    pl.ds(wid * 8, 8),
                    pl.ds(0, 128),
                ],
                osem,
            )

        def compute(ci, p):
            rows_v = rows_b[p]
            idx_v = idx_b[p]
            pos_v = pos_b[p]

            def t_body(t, carry):
                l = t // BSUB
                bs = t % BSUB
                pos4 = [pos_v[l, pl.ds(LANES * k, LANES)]
                        for k in range(NVEC)]
                tokv = plsc.load_gather(
                    idx_v, [bs * (LANES * LCHUNK) + lane * LCHUNK + l]
                )
                maskv = jnp.where(tokv != PAD_IDX, 1.0, 0.0)
                rowvs = [
                    jnp.full((LANES,), l * 8 + 2 * k, dtype=jnp.int32)
                    + iota_div8
                    for k in range(NVEC)
                ]

                for i in range(LANES):
                    r = (bs * LANES + i) * LCHUNK + l
                    x = [rows_v[r, pl.ds(LANES * k, LANES)] + pos4[k]
                         for k in range(NVEC)]
                    s = (x[0] + x[1]) + (x[2] + x[3])
                    q = (x[0] * x[0] + x[1] * x[1]) + (x[2] * x[2] + x[3] * x[3])
                    mean = jnp.sum(s) * INV_EMBED
                    var = jnp.sum(q) * INV_EMBED - mean * mean
                    z = jnp.full((LANES,), var + EPS, dtype=jnp.float32)
                    rstd = _rsqrt(z)
                    mf = maskv[i]
                    a = rstd * mf
                    iv = jnp.full((LANES,), bs * LANES + i, dtype=jnp.int32)
                    for k in range(NVEC):
                        y = (x[k] - mean) * a * g4[k] + b4[k] * mf
                        plsc.store_scatter(
                            ostage, [rowvs[k], iota_mod8, iv], y
                        )
                return carry
            lax.fori_loop(0, LCHUNK * BSUB, t_body, 0)

        stage_and_gather(0, 0)

        def pair_body(h, carry):
            for p in (0, 1):
                ci = h * 2 + p
                wait_gather(ci, p)

                @pl.when(ci + 1 < NCHUNKS)
                def _prep():
                    stage_and_gather(ci + 1, 1 - p)

                @pl.when(ci > 0)
                def _drain():
                    out_copy(ci - 1).wait()

                compute(ci, p)
                out_copy(ci).start()
            return carry

        lax.fori_loop(0, NCHUNKS // 2, pair_body, 0)
        out_copy(NCHUNKS - 1).wait()

    return emb_kernel


_EMB_KERNEL = _make_kernel()


def kernel(tokens, W_word, W_pos, ln_gamma, ln_beta):
    tok = tokens.astype(jnp.int32)
    tok_r = (
        tok.reshape(NW, B_PER_W, NCHUNKS, LCHUNK)
        .transpose(0, 2, 1, 3)
        .reshape(-1)
    )
    out3 = _EMB_KERNEL(tok_r, W_word, W_pos, ln_gamma, ln_beta)
    return (
        out3.reshape(MAX_LEN, DT, NW, 8, 128)
        .transpose(2, 4, 0, 1, 3)
        .reshape(BATCH, MAX_LEN, EMBED)
    )
